# baseline (speedup 1.0000x reference)
import jax
import jax.numpy as jnp
from jax import lax
from jax.experimental import pallas as pl
from jax.experimental.pallas import tpu as pltpu
from jax.experimental.pallas import tpu_sc as plsc

D = 64
SCALE = 8.0
B_TOKENS = 16384
SEQ = 50
VOCAB = 1000000
NC = 2
NS = 16
NW = NC * NS
BW = B_TOKENS // NW
R = 256
N_VEC = R * D // 16


def _transpose_scale(gbuf, stage):
    lane = jax.lax.iota(jnp.int32, 16)
    row_step = lane * D

    @plsc.parallel_loop(0, N_VEC, 1, unroll=8)
    def _(i):
        dr = i >> 7
        j = (i >> 6) & 1
        d8 = (i >> 3) & 7
        b16 = (i & 7) * 16
        col = dr * 8 + d8
        tok = j * 128 + b16
        rows = tok + (row_step // D)
        cols = jnp.full((16,), col, dtype=jnp.int32)
        vec = plsc.load_gather(gbuf, [rows, cols])
        stage[dr, j, d8, pl.ds(b16, 16)] = vec * SCALE


def _emb_body(xT_hbm, lut_hbm, out_hbm, idx_v, g0, g1, st0, st1,
              gsem0, gsem1, osem0, osem1):
    wid = lax.axis_index("s") * NC + lax.axis_index("c")
    b0 = wid * BW
    bc0 = wid * 4

    pltpu.sync_copy(xT_hbm.at[:, pl.ds(b0, BW)], idx_v)

    def gather(s, h, gbuf, gsem):
        idx_sl = idx_v.at[s, pl.ds(h * R, R)]
        pltpu.make_async_copy(lut_hbm.at[idx_sl], gbuf, gsem).start()

    def out_desc(s, h, stage, osem):
        dst = out_hbm.at[s, :, pl.ds(bc0 + 2 * h, 2), :, :]
        return pltpu.make_async_copy(stage, dst, osem)

    gather(0, 0, g0, gsem0)

    def pair(p, carry):
        pltpu.make_async_copy(lut_hbm.at[idx_v.at[p, pl.ds(0, R)]], g0,
                              gsem0).wait()
        gather(p, 1, g1, gsem1)

        @pl.when(p >= 1)
        def _():
            out_desc(p - 1, 0, st0, osem0).wait()

        _transpose_scale(g0, st0)
        out_desc(p, 0, st0, osem0).start()

        pltpu.make_async_copy(lut_hbm.at[idx_v.at[p, pl.ds(R, R)]], g1,
                              gsem1).wait()

        @pl.when(p < SEQ - 1)
        def _():
            gather(p + 1, 0, g0, gsem0)

        @pl.when(p >= 1)
        def _():
            out_desc(p - 1, 1, st1, osem1).wait()

        _transpose_scale(g1, st1)
        out_desc(p, 1, st1, osem1).start()
        return carry

    lax.fori_loop(0, SEQ, pair, 0)

    out_desc(SEQ - 1, 0, st0, osem0).wait()
    out_desc(SEQ - 1, 1, st1, osem1).wait()


def kernel(x, lut):
    xT = jnp.swapaxes(x, 0, 1).astype(jnp.int32)
    mesh = plsc.VectorSubcoreMesh(core_axis_name="c", subcore_axis_name="s")
    out5 = pl.kernel(
        _emb_body,
        mesh=mesh,
        out_type=jax.ShapeDtypeStruct((SEQ, 8, 128, 8, 128), jnp.float32),
        scratch_types=[
            pltpu.VMEM((SEQ, BW), jnp.int32),
            pltpu.VMEM((R, D), jnp.float32),
            pltpu.VMEM((R, D), jnp.float32),
            pltpu.VMEM((8, 2, 8, 128), jnp.float32),
            pltpu.VMEM((8, 2, 8, 128), jnp.float32),
            pltpu.SemaphoreType.DMA,
            pltpu.SemaphoreType.DMA,
            pltpu.SemaphoreType.DMA,
            pltpu.SemaphoreType.DMA,
        ],
        compiler_params=pltpu.CompilerParams(
            use_tc_tiling_on_sc=False, needs_layout_passes=False
        ),
    )(xT, lut)
    out = jnp.transpose(out5, (2, 4, 0, 1, 3))
    return jnp.reshape(out, (B_TOKENS, SEQ, D))


# speedup vs baseline: 1.2113x; 1.2113x over previous
import jax
import jax.numpy as jnp
from jax import lax
from jax.experimental import pallas as pl
from jax.experimental.pallas import tpu as pltpu
from jax.experimental.pallas import tpu_sc as plsc

D = 64
SCALE = 8.0
B_TOKENS = 16384
SEQ = 50
NC = 2
NS = 16
NW = NC * NS
BW = B_TOKENS // NW
R = 256
OUT_ELEMS = B_TOKENS * SEQ * D
S_STRIDE = D * B_TOKENS
DR_STRIDE = 8 * B_TOKENS
SEG = 2 * 8 * 128


def _retile_scale(gbuf, stage, lane_off):
    for j in range(2):
        for g in range(4):
            base = g * 4096 + j * 1024

            @plsc.parallel_loop(0, 128, 1, unroll=8)
            def _(t):
                vec = gbuf[j * 128 + t, pl.ds(g * 16, 16)]
                offs = lane_off + (base + t)
                plsc.store_scatter(stage, [offs], vec * SCALE)


def _emb_body(xT_hbm, lut_hbm, out_hbm, idx_v, g0, g1, st0, st1,
              gsem0, gsem1, osem0, osem1):
    wid = lax.axis_index("s") * NC + lax.axis_index("c")
    b0 = wid * BW
    bc0 = wid * 4

    lane = jax.lax.iota(jnp.int32, 16)
    lane_off = (lane >> 3) * 2048 + (lane & 7) * 128

    pltpu.sync_copy(xT_hbm.at[:, pl.ds(b0, BW)], idx_v)

    def gather(s, h, gbuf, gsem):
        idx_sl = idx_v.at[s, pl.ds(h * R, R)]
        pltpu.make_async_copy(lut_hbm.at[idx_sl], gbuf, gsem).start()

    def out_descs(s, h, stage, osem):
        base = s * S_STRIDE + (bc0 + 2 * h) * 1024
        return [
            pltpu.make_async_copy(
                stage.at[pl.ds(dr * SEG, SEG)],
                out_hbm.at[pl.ds(base + dr * DR_STRIDE, SEG)],
                osem,
            )
            for dr in range(8)
        ]

    gather(0, 0, g0, gsem0)

    def pair(p, carry):
        pltpu.make_async_copy(lut_hbm.at[idx_v.at[p, pl.ds(0, R)]], g0,
                              gsem0).wait()
        gather(p, 1, g1, gsem1)

        @pl.when(p >= 1)
        def _():
            for c in out_descs(p - 1, 0, st0, osem0):
                c.wait()

        _retile_scale(g0, st0, lane_off)
        for c in out_descs(p, 0, st0, osem0):
            c.start()

        pltpu.make_async_copy(lut_hbm.at[idx_v.at[p, pl.ds(R, R)]], g1,
                              gsem1).wait()

        @pl.when(p < SEQ - 1)
        def _():
            gather(p + 1, 0, g0, gsem0)

        @pl.when(p >= 1)
        def _():
            for c in out_descs(p - 1, 1, st1, osem1):
                c.wait()

        _retile_scale(g1, st1, lane_off)
        for c in out_descs(p, 1, st1, osem1):
            c.start()
        return carry

    lax.fori_loop(0, SEQ, pair, 0)

    for c in out_descs(SEQ - 1, 0, st0, osem0):
        c.wait()
    for c in out_descs(SEQ - 1, 1, st1, osem1):
        c.wait()


def kernel(x, lut):
    xT = jnp.swapaxes(x, 0, 1).astype(jnp.int32)
    mesh = plsc.VectorSubcoreMesh(core_axis_name="c", subcore_axis_name="s")
    out_flat = pl.kernel(
        _emb_body,
        mesh=mesh,
        out_type=jax.ShapeDtypeStruct((OUT_ELEMS,), jnp.float32),
        scratch_types=[
            pltpu.VMEM((SEQ, BW), jnp.int32),
            pltpu.VMEM((R, D), jnp.float32),
            pltpu.VMEM((R, D), jnp.float32),
            pltpu.VMEM((R * D,), jnp.float32),
            pltpu.VMEM((R * D,), jnp.float32),
            pltpu.SemaphoreType.DMA,
            pltpu.SemaphoreType.DMA,
            pltpu.SemaphoreType.DMA,
            pltpu.SemaphoreType.DMA,
        ],
        compiler_params=pltpu.CompilerParams(
            use_tc_tiling_on_sc=False, needs_layout_passes=False
        ),
    )(xT, lut)
    out5 = jnp.reshape(out_flat, (SEQ, 8, 128, 8, 128))
    out = jnp.transpose(out5, (2, 4, 0, 1, 3))
    return jnp.reshape(out, (B_TOKENS, SEQ, D))


# speedup vs baseline: 2.0526x; 1.6945x over previous
import jax
import jax.numpy as jnp
from jax import lax
from jax.experimental import pallas as pl
from jax.experimental.pallas import tpu as pltpu
from jax.experimental.pallas import tpu_sc as plsc

D = 64
SCALE = 8.0
B_TOKENS = 16384
SEQ = 50
NC = 2
NS = 16
NW = NC * NS
BW = B_TOKENS // NW
R = 256
OUT_ELEMS = B_TOKENS * SEQ * D
S_STRIDE = D * B_TOKENS
DR_STRIDE = 8 * B_TOKENS
SEG = 2 * 8 * 128


def _build_diag_tables(colv_tab, offv_tab):
    lane = jax.lax.iota(jnp.int32, 16)

    @plsc.parallel_loop(0, D, 1, unroll=2)
    def _(c0):
        cc = (c0 + lane) & (D - 1)
        colv_tab[pl.ds(c0 * 16, 16)] = cc
        offv_tab[pl.ds(c0 * 16, 16)] = ((cc >> 3) << 11) + ((cc & 7) << 7) + lane


def _retile_scale(gbuf, stage, colv_tab, offv_tab):
    lane = jax.lax.iota(jnp.int32, 16)
    for j in range(2):

        @plsc.parallel_loop(0, D, 1, unroll=2)
        def _(c0):
            colv = colv_tab[pl.ds(c0 * 16, 16)]
            offv = offv_tab[pl.ds(c0 * 16, 16)]
            for t0 in range(0, 128, 16):
                rows = lane + (j * 128 + t0)
                vec = plsc.load_gather(gbuf, [rows, colv])
                offs = offv + (j * 1024 + t0)
                plsc.store_scatter(stage, [offs], vec * SCALE)


def _emb_body(xT_hbm, lut_hbm, out_hbm, idx_v, g0, g1, st0, st1, colv_tab,
              offv_tab, gsem0, gsem1, osem0, osem1):
    wid = lax.axis_index("s") * NC + lax.axis_index("c")
    b0 = wid * BW
    bc0 = wid * 4

    _build_diag_tables(colv_tab, offv_tab)

    pltpu.sync_copy(xT_hbm.at[:, pl.ds(b0, BW)], idx_v)

    def gather(s, h, gbuf, gsem):
        idx_sl = idx_v.at[s, pl.ds(h * R, R)]
        pltpu.make_async_copy(
            lut_hbm.at[idx_sl], gbuf, gsem
        ).start()

    def out_descs(s, h, stage, osem):
        base = s * S_STRIDE + (bc0 + 2 * h) * 1024
        return [
            pltpu.make_async_copy(
                stage.at[pl.ds(dr * SEG, SEG)],
                out_hbm.at[pl.ds(base + dr * DR_STRIDE, SEG)],
                osem,
            )
            for dr in range(8)
        ]

    gather(0, 0, g0, gsem0)

    def pair(p, carry):
        pltpu.make_async_copy(lut_hbm.at[idx_v.at[p, pl.ds(0, R)]],
                              g0, gsem0).wait()
        gather(p, 1, g1, gsem1)

        @pl.when(p >= 1)
        def _():
            for c in out_descs(p - 1, 0, st0, osem0):
                c.wait()

        _retile_scale(g0, st0, colv_tab, offv_tab)
        for c in out_descs(p, 0, st0, osem0):
            c.start()

        pltpu.make_async_copy(lut_hbm.at[idx_v.at[p, pl.ds(R, R)]],
                              g1, gsem1).wait()

        @pl.when(p < SEQ - 1)
        def _():
            gather(p + 1, 0, g0, gsem0)

        @pl.when(p >= 1)
        def _():
            for c in out_descs(p - 1, 1, st1, osem1):
                c.wait()

        _retile_scale(g1, st1, colv_tab, offv_tab)
        for c in out_descs(p, 1, st1, osem1):
            c.start()
        return carry

    lax.fori_loop(0, SEQ, pair, 0)

    for c in out_descs(SEQ - 1, 0, st0, osem0):
        c.wait()
    for c in out_descs(SEQ - 1, 1, st1, osem1):
        c.wait()


def kernel(x, lut):
    xT = jnp.swapaxes(x, 0, 1).astype(jnp.int32)
    mesh = plsc.VectorSubcoreMesh(core_axis_name="c", subcore_axis_name="s")
    out_flat = pl.kernel(
        _emb_body,
        mesh=mesh,
        out_type=jax.ShapeDtypeStruct((OUT_ELEMS,), jnp.float32),
        scratch_types=[
            pltpu.VMEM((SEQ, BW), jnp.int32),
            pltpu.VMEM((R, D), jnp.float32),
            pltpu.VMEM((R, D), jnp.float32),
            pltpu.VMEM((R * D,), jnp.float32),
            pltpu.VMEM((R * D,), jnp.float32),
            pltpu.VMEM((D * 16,), jnp.int32),
            pltpu.VMEM((D * 16,), jnp.int32),
            pltpu.SemaphoreType.DMA,
            pltpu.SemaphoreType.DMA,
            pltpu.SemaphoreType.DMA,
            pltpu.SemaphoreType.DMA,
        ],
        compiler_params=pltpu.CompilerParams(
            use_tc_tiling_on_sc=False, needs_layout_passes=False
        ),
    )(xT, lut)
    out5 = jnp.reshape(out_flat, (SEQ, 8, 128, 8, 128))
    out = jnp.transpose(out5, (2, 4, 0, 1, 3))
    return jnp.reshape(out, (B_TOKENS, SEQ, D))


# speedup vs baseline: 2.0548x; 1.0011x over previous
import jax
import jax.numpy as jnp
from jax import lax
from jax.experimental import pallas as pl
from jax.experimental.pallas import tpu as pltpu
from jax.experimental.pallas import tpu_sc as plsc

D = 64
SCALE = 8.0
B_TOKENS = 16384
SEQ = 50
NC = 2
NS = 16
NW = NC * NS
BW = B_TOKENS // NW
R = 256
OUT_ELEMS = B_TOKENS * SEQ * D
S_STRIDE = D * B_TOKENS
DR_STRIDE = 8 * B_TOKENS
SEG = 2 * 8 * 128


def _build_diag_tables(colv_tab, offv_tab):
    lane = jax.lax.iota(jnp.int32, 16)

    @plsc.parallel_loop(0, D, 1, unroll=2)
    def _(c0):
        cc = (c0 + lane) & (D - 1)
        colv_tab[pl.ds(c0 * 16, 16)] = cc
        offv_tab[pl.ds(c0 * 16, 16)] = ((cc >> 3) << 11) + ((cc & 7) << 7) + lane


def _retile_scale(gbuf, stage, colv_tab, offv_tab):
    lane = jax.lax.iota(jnp.int32, 16)
    for j in range(2):

        @plsc.parallel_loop(0, D, 1, unroll=2)
        def _(c0):
            colv = colv_tab[pl.ds(c0 * 16, 16)]
            offv = offv_tab[pl.ds(c0 * 16, 16)]
            for t0 in range(0, 128, 16):
                rows = lane + (j * 128 + t0)
                vec = plsc.load_gather(gbuf, [rows, colv])
                offs = offv + (j * 1024 + t0)
                plsc.store_scatter(stage, [offs], vec * SCALE)


def _emb_body(xT_hbm, lut_hbm, out_hbm, idx_v, g0, g1, st0, st1, colv_tab,
              offv_tab, gsem0, gsem1, osem0, osem1):
    wid = lax.axis_index("s") * NC + lax.axis_index("c")
    b0 = wid * BW
    bc0 = wid * 4

    _build_diag_tables(colv_tab, offv_tab)

    pltpu.sync_copy(xT_hbm.at[:, pl.ds(b0, BW)], idx_v)

    def gather(s, h, gbuf, gsem):
        idx_sl = idx_v.at[s, pl.ds(h * R, R)]
        pltpu.make_async_copy(
            lut_hbm.at[idx_sl], gbuf, gsem
        ).start()

    def out_descs(s, h, stage, osem):
        base = s * S_STRIDE + (bc0 + 2 * h) * 1024
        return [
            pltpu.make_async_copy(
                stage.at[pl.ds(dr * SEG, SEG)],
                out_hbm.at[pl.ds(base + dr * DR_STRIDE, SEG)],
                osem,
            )
            for dr in range(8)
        ]

    gather(0, 0, g0, gsem0)

    def pair(p, carry):
        pltpu.make_async_copy(lut_hbm.at[idx_v.at[p, pl.ds(0, R)]],
                              g0, gsem0).wait()
        gather(p, 1, g1, gsem1)

        @pl.when(p >= 1)
        def _():
            for c in out_descs(p - 1, 0, st0, osem0):
                c.wait()

        _retile_scale(g0, st0, colv_tab, offv_tab)
        for c in out_descs(p, 0, st0, osem0):
            c.start()

        pltpu.make_async_copy(lut_hbm.at[idx_v.at[p, pl.ds(R, R)]],
                              g1, gsem1).wait()

        @pl.when(p < SEQ - 1)
        def _():
            gather(p + 1, 0, g0, gsem0)

        @pl.when(p >= 1)
        def _():
            for c in out_descs(p - 1, 1, st1, osem1):
                c.wait()

        _retile_scale(g1, st1, colv_tab, offv_tab)
        for c in out_descs(p, 1, st1, osem1):
            c.start()
        return carry

    lax.fori_loop(0, SEQ, pair, 0)

    for c in out_descs(SEQ - 1, 0, st0, osem0):
        c.wait()
    for c in out_descs(SEQ - 1, 1, st1, osem1):
        c.wait()


def kernel(x, lut):
    lut = jnp.reshape(
        lax.optimization_barrier(jnp.reshape(lut, (1000000 * D,))),
        (1000000, D),
    )
    xT = jnp.swapaxes(x, 0, 1).astype(jnp.int32)
    mesh = plsc.VectorSubcoreMesh(core_axis_name="c", subcore_axis_name="s")
    out_flat = pl.kernel(
        _emb_body,
        mesh=mesh,
        out_type=jax.ShapeDtypeStruct((OUT_ELEMS,), jnp.float32),
        scratch_types=[
            pltpu.VMEM((SEQ, BW), jnp.int32),
            pltpu.VMEM((R, D), jnp.float32),
            pltpu.VMEM((R, D), jnp.float32),
            pltpu.VMEM((R * D,), jnp.float32),
            pltpu.VMEM((R * D,), jnp.float32),
            pltpu.VMEM((D * 16,), jnp.int32),
            pltpu.VMEM((D * 16,), jnp.int32),
            pltpu.SemaphoreType.DMA,
            pltpu.SemaphoreType.DMA,
            pltpu.SemaphoreType.DMA,
            pltpu.SemaphoreType.DMA,
        ],
        compiler_params=pltpu.CompilerParams(
            use_tc_tiling_on_sc=False, needs_layout_passes=False
        ),
    )(xT, lut)
    out5 = jnp.reshape(out_flat, (SEQ, 8, 128, 8, 128))
    out = jnp.transpose(out5, (2, 4, 0, 1, 3))
    return jnp.reshape(out, (B_TOKENS, SEQ, D))
